# Initial kernel scaffold; baseline (speedup 1.0000x reference)
import jax
import jax.numpy as jnp
from jax.experimental import pallas as pl


def kernel(x, embedding, fc_table, w_lin, b_lin, W1, b1, W2, b2, W3, b3):
    raise NotImplementedError("write your pallas kernel here")


# speedup vs baseline: 1.1746x; 1.1746x over previous
import functools

import jax
import jax.numpy as jnp
from jax import lax
from jax.experimental import pallas as pl
from jax.experimental.pallas import tpu as pltpu
from jax.experimental.pallas import tpu_sc as plsc

B = 16384
F = 26
K = 16
EMBED_OUT = F * K
BF = B * F

NC = 2
NS = 16
NW = NC * NS
PER_W = BF // NW
CHUNK = 128
NCHUNK = PER_W // CHUNK

def _sc_gather_body(idx_hbm, emb_hbm, fct_hbm, rows_hbm, fcv_hbm,
                    idx_v, rows_v, fcv_v, sem_e, sem_f):
    wid = lax.axis_index("s") * NC + lax.axis_index("c")
    base = wid * PER_W
    pltpu.sync_copy(idx_hbm.at[pl.ds(base, PER_W)], idx_v)

    @pl.loop(0, NCHUNK)
    def _chunk(c):
        off = c * CHUNK
        idxs = idx_v.at[pl.ds(off, CHUNK)]
        cp_e = pltpu.async_copy(emb_hbm.at[idxs], rows_v, sem_e)
        cp_f = pltpu.async_copy(fct_hbm.at[idxs], fcv_v, sem_f)
        cp_e.wait()
        cp_f.wait()
        pltpu.sync_copy(rows_v, rows_hbm.at[pl.ds(base + off, CHUNK)])
        pltpu.sync_copy(fcv_v, fcv_hbm.at[pl.ds(base + off, CHUNK)])


_SC_OUT_TYPE = (
    jax.ShapeDtypeStruct((BF, K), jnp.float32),
    jax.ShapeDtypeStruct((BF,), jnp.float32),
)
_SC_SCRATCH = (
    pltpu.VMEM((PER_W,), jnp.int32),
    pltpu.VMEM((CHUNK, K), jnp.float32),
    pltpu.VMEM((CHUNK,), jnp.float32),
    pltpu.SemaphoreType.DMA,
    pltpu.SemaphoreType.DMA,
)


@functools.cache
def _build_sc_gather():
    mesh = plsc.VectorSubcoreMesh(
        core_axis_name="c", subcore_axis_name="s",
        num_cores=NC, num_subcores=NS,
    )
    return pl.kernel(
        _sc_gather_body,
        out_type=_SC_OUT_TYPE,
        mesh=mesh,
        compiler_params=pltpu.CompilerParams(use_tc_tiling_on_sc=False),
        scratch_types=_SC_SCRATCH,
    )


BLK = 1024


def _mlp_body(e_ref, fc_ref, s_ref, w1_ref, b1_ref, w2_ref, b2_ref,
              w3_ref, b3_ref, wlin_ref, blin_ref, out_ref):
    e = e_ref[...]
    s = s_ref[...]
    sum_f = lax.dot_general(e, s, (((1,), (0,)), ((), ())),
                            preferred_element_type=jnp.float32)
    ssq = lax.dot_general(e * e, s, (((1,), (0,)), ((), ())),
                          preferred_element_type=jnp.float32)
    fm = 0.5 * jnp.sum(sum_f * sum_f - ssq, axis=1, keepdims=True)

    lin = jnp.sum(fc_ref[...], axis=1, keepdims=True)
    lin = lin * wlin_ref[0, 0] + blin_ref[0, 0]

    h = lax.dot_general(e, w1_ref[...], (((1,), (0,)), ((), ())),
                        preferred_element_type=jnp.float32)
    h = jnp.maximum(h + b1_ref[...], 0.0)
    h = lax.dot_general(h, w2_ref[...], (((1,), (0,)), ((), ())),
                        preferred_element_type=jnp.float32)
    h = jnp.maximum(h + b2_ref[...], 0.0)
    mlp = lax.dot_general(h, w3_ref[...], (((1,), (0,)), ((), ())),
                          preferred_element_type=jnp.float32)
    mlp = mlp + b3_ref[0, 0]

    z = lin + fm + mlp
    out_ref[...] = 1.0 / (1.0 + jnp.exp(-z))


def _tc_mlp(e, fcm, s, w1, b1, w2, b2, w3, b3, wlin, blin):
    grid = (B // BLK,)
    fixed = lambda i: (0, 0)
    return pl.pallas_call(
        _mlp_body,
        grid=grid,
        in_specs=[
            pl.BlockSpec((BLK, EMBED_OUT), lambda i: (i, 0)),
            pl.BlockSpec((BLK, F), lambda i: (i, 0)),
            pl.BlockSpec((EMBED_OUT, K), fixed),
            pl.BlockSpec((EMBED_OUT, 128), fixed),
            pl.BlockSpec((1, 128), fixed),
            pl.BlockSpec((128, 64), fixed),
            pl.BlockSpec((1, 64), fixed),
            pl.BlockSpec((64, 1), fixed),
            pl.BlockSpec((1, 1), fixed),
            pl.BlockSpec((1, 1), fixed),
            pl.BlockSpec((1, 1), fixed),
        ],
        out_specs=pl.BlockSpec((BLK, 1), lambda i: (i, 0)),
        out_shape=jax.ShapeDtypeStruct((B, 1), jnp.float32),
    )(e, fcm, s, w1, b1, w2, b2, w3, b3, wlin, blin)


def kernel(x, embedding, fc_table, w_lin, b_lin, W1, b1, W2, b2, W3, b3):
    xf = x.reshape(-1).astype(jnp.int32)
    rows, fcv = _build_sc_gather()(xf, embedding, fc_table.reshape(-1))
    e = rows.reshape(B, EMBED_OUT)
    fcm = fcv.reshape(B, F)
    s = (jnp.arange(EMBED_OUT)[:, None] % K == jnp.arange(K)[None, :]
         ).astype(jnp.float32)
    return _tc_mlp(e, fcm, s, W1, b1.reshape(1, -1), W2, b2.reshape(1, -1),
                   W3, b3.reshape(1, 1), w_lin, b_lin.reshape(1, 1))
